# baseline (speedup 1.0000x reference)
import jax
import jax.numpy as jnp
from jax import lax
from jax.experimental import pallas as pl
from jax.experimental.pallas import tpu as pltpu
from jax.experimental.pallas import tpu_sc as plsc

N = 10000
E = 320000
D = 128
SLOPE = 0.2
NC = 2
NS = 16
NW = NC * NS
C = 64
NCHUNK = E // C
TMAX = 158
NPAD = 10240
EPS = 1e-9


def _k1_body(x_ref, ws_ref, bs_ref, wd_ref, bd_ref, attn_ref, wel_ref,
             fs_ref, fd_ref, v_ref, fd16_ref):
    xb = x_ref[...]
    fs_ref[...] = lax.dot_general(xb, ws_ref[...], (((1,), (1,)), ((), ())),
                                  preferred_element_type=jnp.float32) + bs_ref[...]
    fd_ref[...] = lax.dot_general(xb, wd_ref[...], (((1,), (1,)), ((), ())),
                                  preferred_element_type=jnp.float32) + bd_ref[...]

    fd = fd_ref[...]
    blk = fd.shape[0]
    fd16_ref[...] = fd.reshape(blk, 4, 2, 16).swapaxes(2, 3).reshape(
        blk, D).astype(jnp.bfloat16)

    @pl.when(pl.program_id(0) == 0)
    def _():
        v_ref[...] = lax.dot_general(attn_ref[...], wel_ref[...],
                                     (((1,), (0,)), ((), ())),
                                     preferred_element_type=jnp.float32)


def _k1(x, W_src, b_src, W_dst, b_dst, attn_r, W_el):
    blk = 1000
    grid = N // blk
    full = lambda i: (0, 0)
    return pl.pallas_call(
        _k1_body,
        grid=(grid,),
        in_specs=[
            pl.BlockSpec((blk, D), lambda i: (i, 0)),
            pl.BlockSpec((D, D), full),
            pl.BlockSpec((1, D), full),
            pl.BlockSpec((D, D), full),
            pl.BlockSpec((1, D), full),
            pl.BlockSpec((1, D), full),
            pl.BlockSpec((D, D), full),
        ],
        out_specs=[
            pl.BlockSpec((blk, D), lambda i: (i, 0)),
            pl.BlockSpec((blk, D), lambda i: (i, 0)),
            pl.BlockSpec((1, D), full),
            pl.BlockSpec((blk, D), lambda i: (i, 0)),
        ],
        out_shape=[
            jax.ShapeDtypeStruct((N, D), jnp.float32),
            jax.ShapeDtypeStruct((N, D), jnp.float32),
            jax.ShapeDtypeStruct((1, D), jnp.float32),
            jax.ShapeDtypeStruct((N, D), jnp.bfloat16),
        ],
    )(x, W_src, b_src, W_dst, b_dst, attn_r, W_el)


def _k2_body(fs_hbm, fd_hbm, v_hbm, src_hbm, dst_hbm,
             op_hbm, dden_hbm,
             idx_s0, idx_d0, idx_s1, idx_d1, idd0, idd1,
             el0, er0, el1, er1, sb, scz0, scz1, vv,
             den_sh, out_sh, sem0, sem1, sem_sc,
             sem_i0, sem_i1, sem_z0, sem_z1):
    cid = lax.axis_index("c")
    sid = lax.axis_index("s")
    wid = sid * NC + cid
    nch = (NCHUNK - 1 - wid) // NW + 1
    lane = lax.iota(jnp.int32, 16)

    bufs = ((idx_s0, idx_d0, idd0, el0, er0, scz0, sem0, sem_i0, sem_z0),
            (idx_s1, idx_d1, idd1, el1, er1, scz1, sem1, sem_i1, sem_z1))

    def issue_idx(t, b, sync):
        idx_s, idx_d = bufs[b][0], bufs[b][1]
        sem_i = bufs[b][7]
        real = t < nch
        base = jnp.where(real, wid + t * NW, wid) * C
        if sync:
            pltpu.sync_copy(src_hbm.at[pl.ds(base, C)], idx_s)
            pltpu.sync_copy(dst_hbm.at[pl.ds(base, C)], idx_d)
        else:
            pltpu.async_copy(src_hbm.at[pl.ds(base, C)], idx_s, sem_i)
            pltpu.async_copy(dst_hbm.at[pl.ds(base, C)], idx_d, sem_i)

    def wait_idx(b):
        idx_s, idx_d = bufs[b][0], bufs[b][1]
        sem_i = bufs[b][7]
        pltpu.make_async_copy(src_hbm.at[pl.ds(0, C)], idx_s, sem_i).wait()
        pltpu.make_async_copy(dst_hbm.at[pl.ds(0, C)], idx_d, sem_i).wait()

    def issue_gather(b):
        idx_s, idx_d, _, el, er, _, sem = bufs[b][:7]
        pltpu.async_copy(fs_hbm.at[idx_s], el, sem)
        pltpu.async_copy(fd_hbm.at[idx_d], er, sem)

    def wait_gather(b):
        idx_s, idx_d, _, el, er, _, sem = bufs[b][:7]
        pltpu.make_async_copy(fs_hbm.at[idx_s], el, sem).wait()
        pltpu.make_async_copy(fd_hbm.at[idx_d], er, sem).wait()

    def wait_rowscatter():
        pltpu.make_async_copy(sb, out_sh.at[idd0], sem_sc).wait()

    def wait_zscatter(b):
        scz = bufs[b][5]
        pltpu.make_async_copy(scz, den_sh.at[idd0], bufs[b][8]).wait()

    @pl.loop(0, C // 16)
    def _(i):
        scz0[pl.ds(i * 16, 16)] = jnp.zeros((16,), jnp.float32)
        scz1[pl.ds(i * 16, 16)] = jnp.zeros((16,), jnp.float32)
        idd0[pl.ds(i * 16, 16)] = jnp.zeros((16,), jnp.int32)

    @pl.loop(0, C)
    def _(e):
        for k in range(8):
            sb[e, pl.ds(k * 16, 16)] = jnp.zeros((16,), jnp.float32)

    for t in range(10):
        pltpu.sync_copy(scz0, den_sh.at[pl.ds(sid * 640 + t * C, C)])
        pltpu.sync_copy(sb, out_sh.at[pl.ds(sid * 640 + t * C, C)])
    pltpu.sync_copy(v_hbm, vv)

    issue_idx(0, 0, True)
    issue_idx(1, 1, False)
    issue_gather(0)
    pltpu.async_copy(sb, out_sh.at[idd0], sem_sc, add=True)
    pltpu.async_copy(scz0, den_sh.at[idd0], sem_z0, add=True)
    pltpu.async_copy(scz1, den_sh.at[idd0], sem_z1, add=True)
    plsc.subcore_barrier()

    vvr = [vv[pl.ds(k * 16, 16)] for k in range(8)]

    def phase(t, b):
        idx_s, idx_d, idd, el, er, scz, sem, sem_i, sem_z = bufs[b]
        ob = 1 - b
        wait_gather(b)
        wait_idx(ob)
        issue_gather(ob)
        wait_zscatter(b)

        @pl.loop(0, C // 16)
        def _(i):
            idd[pl.ds(i * 16, 16)] = idx_d[pl.ds(i * 16, 16)]
        issue_idx(t + 2, b, False)
        wait_rowscatter()
        valid = jnp.where(t < nch, 1.0, 0.0)

        @pl.loop(0, C // 16)
        def _(g):
            szv = jnp.zeros((16,), jnp.float32)
            for j in range(16):
                e = g * 16 + j
                lrow = [el[e, pl.ds(k * 16, 16)] for k in range(8)]
                a4 = [jnp.zeros((16,), jnp.float32) for _ in range(4)]
                for m in range(4):
                    w = er[e, pl.ds(m * 16, 16)]
                    ra = lax.bitcast_convert_type(w << 16, jnp.float32)
                    rb = lax.bitcast_convert_type(
                        w & jnp.int32(-65536), jnp.float32)
                    for k, rr in ((2 * m, ra), (2 * m + 1, rb)):
                        z = lrow[k] + rr
                        z = jnp.maximum(z, SLOPE * z)
                        a4[k % 4] = a4[k % 4] + z * vvr[k]
                acc = (a4[0] + a4[1]) + (a4[2] + a4[3])
                for sh in (8, 4, 2, 1):
                    acc = acc + acc.at[lane ^ sh].get(mode="promise_in_bounds")
                pe = jnp.exp(acc) * valid
                for k in range(8):
                    sb[e, pl.ds(k * 16, 16)] = lrow[k] * pe
                szv = jnp.where(lane == j, pe, szv)
            scz[pl.ds(g * 16, 16)] = szv

        pltpu.async_copy(scz, den_sh.at[idd], sem_z, add=True)
        pltpu.async_copy(sb, out_sh.at[idd], sem_sc, add=True)

    @pl.loop(0, TMAX // 2)
    def _(i):
        phase(2 * i, 0)
        phase(2 * i + 1, 1)

    wait_gather(0)
    wait_idx(1)
    wait_rowscatter()
    wait_zscatter(0)
    wait_zscatter(1)
    plsc.subcore_barrier()

    pltpu.sync_copy(den_sh.at[pl.ds(sid * 640, 640)],
                    dden_hbm.at[pl.ds(cid * NPAD + sid * 640, 640)])
    for t in range(10):
        pltpu.sync_copy(out_sh.at[pl.ds(sid * 640 + t * C, C)],
                        op_hbm.at[pl.ds(cid * NPAD + sid * 640 + t * C, C)])


def _k2(fs, fd, v1d, src, dst):
    mesh = plsc.VectorSubcoreMesh(core_axis_name="c", subcore_axis_name="s")
    return pl.kernel(
        _k2_body,
        compiler_params=pltpu.CompilerParams(use_tc_tiling_on_sc=False),
        out_type=[
            jax.ShapeDtypeStruct((NC * NPAD, D), jnp.float32),
            jax.ShapeDtypeStruct((NC * NPAD,), jnp.float32),
        ],
        mesh=mesh,
        scratch_types=[
            pltpu.VMEM((C,), jnp.int32),
            pltpu.VMEM((C,), jnp.int32),
            pltpu.VMEM((C,), jnp.int32),
            pltpu.VMEM((C,), jnp.int32),
            pltpu.VMEM((C,), jnp.int32),
            pltpu.VMEM((C,), jnp.int32),
            pltpu.VMEM((C, D), jnp.float32),
            pltpu.VMEM((C, D // 2), jnp.int32),
            pltpu.VMEM((C, D), jnp.float32),
            pltpu.VMEM((C, D // 2), jnp.int32),
            pltpu.VMEM((C, D), jnp.float32),
            pltpu.VMEM((C,), jnp.float32),
            pltpu.VMEM((C,), jnp.float32),
            pltpu.VMEM((D,), jnp.float32),
            pltpu.VMEM_SHARED((NPAD,), jnp.float32),
            pltpu.VMEM_SHARED((NPAD, D), jnp.float32),
            pltpu.SemaphoreType.DMA,
            pltpu.SemaphoreType.DMA,
            pltpu.SemaphoreType.DMA,
            pltpu.SemaphoreType.DMA,
            pltpu.SemaphoreType.DMA,
            pltpu.SemaphoreType.DMA,
            pltpu.SemaphoreType.DMA,
        ],
    )(fs, fd, v1d, src, dst)


def _k3_body(op_hbm, dden_hbm, fo_hbm,
             a0, a1, d0, d1, inv):
    cid = lax.axis_index("c")
    sid = lax.axis_index("s")
    wid = sid * NC + cid
    r0 = wid * 320

    pltpu.sync_copy(dden_hbm.at[pl.ds(r0, 320)], d0)
    pltpu.sync_copy(dden_hbm.at[pl.ds(NPAD + r0, 320)], d1)

    @pl.loop(0, 20)
    def _(i):
        s = pl.ds(i * 16, 16)
        inv[s] = 1.0 / (d0[s] + d1[s] + EPS)

    for t in range(5):
        rb = r0 + t * 64
        pltpu.sync_copy(op_hbm.at[pl.ds(rb, 64)], a0)
        pltpu.sync_copy(op_hbm.at[pl.ds(NPAD + rb, 64)], a1)

        @pl.loop(0, 4)
        def _(g):
            inv16 = inv[pl.ds(t * 64 + g * 16, 16)]
            for j in range(16):
                e = g * 16 + j
                a = inv16[j]
                for k in range(8):
                    s = pl.ds(k * 16, 16)
                    a0[e, s] = (a0[e, s] + a1[e, s]) * a

        pltpu.sync_copy(a0, fo_hbm.at[pl.ds(rb, 64)])


def _k3(op, dden):
    mesh = plsc.VectorSubcoreMesh(core_axis_name="c", subcore_axis_name="s")
    return pl.kernel(
        _k3_body,
        out_type=jax.ShapeDtypeStruct((NPAD, D), jnp.float32),
        mesh=mesh,
        scratch_types=[
            pltpu.VMEM((64, D), jnp.float32),
            pltpu.VMEM((64, D), jnp.float32),
            pltpu.VMEM((320,), jnp.float32),
            pltpu.VMEM((320,), jnp.float32),
            pltpu.VMEM((320,), jnp.float32),
        ],
    )(op, dden)


def kernel(x, edge_index, W_src, b_src, W_dst, b_dst, attn, W_el, b_el):
    src = edge_index[0]
    dst = edge_index[1]
    attn_r = attn.reshape(1, D)
    fs, fd, v, fd16 = _k1(x, W_src, b_src.reshape(1, D), W_dst,
                          b_dst.reshape(1, D), attn_r, W_el)
    fd16w = lax.bitcast_convert_type(fd16.reshape(N, D // 2, 2), jnp.int32)
    op, dden = _k2(fs, fd16w, v.reshape(D), src, dst)
    fo = _k3(op, dden)
    return fo[:N].reshape(N, 1, D)


# speedup vs baseline: 2.5097x; 2.5097x over previous
import jax
import jax.numpy as jnp
from jax import lax
from jax.experimental import pallas as pl
from jax.experimental.pallas import tpu as pltpu
from jax.experimental.pallas import tpu_sc as plsc

N = 10000
E = 320000
D = 128
SLOPE = 0.2
NC = 2
NS = 16
NW = NC * NS
C = 64
NCHUNK = E // C
TMAX = 158
NPAD = 10240
EPS = 1e-9


def _k1_body(x_ref, ws_ref, bs_ref, wd_ref, bd_ref, attn_ref, wel_ref,
             fs_ref, fd_ref, v_ref):
    xb = x_ref[...]
    fs_ref[...] = lax.dot_general(xb, ws_ref[...], (((1,), (1,)), ((), ())),
                                  preferred_element_type=jnp.float32) + bs_ref[...]
    fd_ref[...] = lax.dot_general(xb, wd_ref[...], (((1,), (1,)), ((), ())),
                                  preferred_element_type=jnp.float32) + bd_ref[...]

    @pl.when(pl.program_id(0) == 0)
    def _():
        v_ref[...] = lax.dot_general(attn_ref[...], wel_ref[...],
                                     (((1,), (0,)), ((), ())),
                                     preferred_element_type=jnp.float32)


def _k1(x, W_src, b_src, W_dst, b_dst, attn_r, W_el):
    blk = 1000
    grid = N // blk
    full = lambda i: (0, 0)
    return pl.pallas_call(
        _k1_body,
        grid=(grid,),
        in_specs=[
            pl.BlockSpec((blk, D), lambda i: (i, 0)),
            pl.BlockSpec((D, D), full),
            pl.BlockSpec((1, D), full),
            pl.BlockSpec((D, D), full),
            pl.BlockSpec((1, D), full),
            pl.BlockSpec((1, D), full),
            pl.BlockSpec((D, D), full),
        ],
        out_specs=[
            pl.BlockSpec((blk, D), lambda i: (i, 0)),
            pl.BlockSpec((blk, D), lambda i: (i, 0)),
            pl.BlockSpec((1, D), full),
        ],
        out_shape=[
            jax.ShapeDtypeStruct((N, D), jnp.float32),
            jax.ShapeDtypeStruct((N, D), jnp.float32),
            jax.ShapeDtypeStruct((1, D), jnp.float32),
        ],
    )(x, W_src, b_src, W_dst, b_dst, attn_r, W_el)


def _k2_body(fs_hbm, fd_hbm, v_hbm, src_hbm, dst_hbm,
             op_hbm, dden_hbm,
             idx_s0, idx_d0, idx_s1, idx_d1, idd0, idd1,
             el0, er0, el1, er1, sb, scz0, scz1, vv,
             den_sh, out_sh, sem0, sem1, sem_sc,
             sem_i0, sem_i1, sem_z0, sem_z1):
    cid = lax.axis_index("c")
    sid = lax.axis_index("s")
    wid = sid * NC + cid
    nch = (NCHUNK - 1 - wid) // NW + 1
    lane = lax.iota(jnp.int32, 16)

    bufs = ((idx_s0, idx_d0, idd0, el0, er0, scz0, sem0, sem_i0, sem_z0),
            (idx_s1, idx_d1, idd1, el1, er1, scz1, sem1, sem_i1, sem_z1))

    def issue_idx(t, b, sync):
        idx_s, idx_d = bufs[b][0], bufs[b][1]
        sem_i = bufs[b][7]
        real = t < nch
        base = jnp.where(real, wid + t * NW, wid) * C
        if sync:
            pltpu.sync_copy(src_hbm.at[pl.ds(base, C)], idx_s)
            pltpu.sync_copy(dst_hbm.at[pl.ds(base, C)], idx_d)
        else:
            pltpu.async_copy(src_hbm.at[pl.ds(base, C)], idx_s, sem_i)
            pltpu.async_copy(dst_hbm.at[pl.ds(base, C)], idx_d, sem_i)

    def wait_idx(b):
        idx_s, idx_d = bufs[b][0], bufs[b][1]
        sem_i = bufs[b][7]
        pltpu.make_async_copy(src_hbm.at[pl.ds(0, C)], idx_s, sem_i).wait()
        pltpu.make_async_copy(dst_hbm.at[pl.ds(0, C)], idx_d, sem_i).wait()

    def issue_gather(b):
        idx_s, idx_d, _, el, er, _, sem = bufs[b][:7]
        pltpu.async_copy(fs_hbm.at[idx_s], el, sem)
        pltpu.async_copy(fd_hbm.at[idx_d], er, sem)

    def wait_gather(b):
        idx_s, idx_d, _, el, er, _, sem = bufs[b][:7]
        pltpu.make_async_copy(fs_hbm.at[idx_s], el, sem).wait()
        pltpu.make_async_copy(fd_hbm.at[idx_d], er, sem).wait()

    def wait_rowscatter():
        pltpu.make_async_copy(sb, out_sh.at[idd0], sem_sc).wait()

    def wait_zscatter(b):
        scz = bufs[b][5]
        pltpu.make_async_copy(scz, den_sh.at[idd0], bufs[b][8]).wait()

    @pl.loop(0, C // 16)
    def _(i):
        scz0[pl.ds(i * 16, 16)] = jnp.zeros((16,), jnp.float32)
        scz1[pl.ds(i * 16, 16)] = jnp.zeros((16,), jnp.float32)
        idd0[pl.ds(i * 16, 16)] = jnp.zeros((16,), jnp.int32)

    @pl.loop(0, C)
    def _(e):
        for k in range(8):
            sb[e, pl.ds(k * 16, 16)] = jnp.zeros((16,), jnp.float32)

    for t in range(10):
        pltpu.sync_copy(scz0, den_sh.at[pl.ds(sid * 640 + t * C, C)])
        pltpu.sync_copy(sb, out_sh.at[pl.ds(sid * 640 + t * C, C)])
    pltpu.sync_copy(v_hbm, vv)

    issue_idx(0, 0, True)
    issue_idx(1, 1, False)
    issue_gather(0)
    pltpu.async_copy(sb, out_sh.at[idd0], sem_sc, add=True)
    pltpu.async_copy(scz0, den_sh.at[idd0], sem_z0, add=True)
    pltpu.async_copy(scz1, den_sh.at[idd0], sem_z1, add=True)
    plsc.subcore_barrier()

    vvr = [vv[pl.ds(k * 16, 16)] for k in range(8)]

    def phase(t, b):
        idx_s, idx_d, idd, el, er, scz, sem, sem_i, sem_z = bufs[b]
        ob = 1 - b
        wait_gather(b)
        wait_idx(ob)
        issue_gather(ob)
        wait_zscatter(b)

        @pl.loop(0, C // 16)
        def _(i):
            idd[pl.ds(i * 16, 16)] = idx_d[pl.ds(i * 16, 16)]
        issue_idx(t + 2, b, False)
        wait_rowscatter()
        valid = jnp.where(t < nch, 1.0, 0.0)

        @pl.loop(0, C // 16)
        def _(g):
            szv = jnp.zeros((16,), jnp.float32)
            for j in range(16):
                e = g * 16 + j
                lrow = [el[e, pl.ds(k * 16, 16)] for k in range(8)]
                a4 = [jnp.zeros((16,), jnp.float32) for _ in range(4)]
                for k in range(8):
                    z = lrow[k] + er[e, pl.ds(k * 16, 16)]
                    z = jnp.maximum(z, SLOPE * z)
                    a4[k % 4] = a4[k % 4] + z * vvr[k]
                acc = (a4[0] + a4[1]) + (a4[2] + a4[3])
                for sh in (8, 4, 2, 1):
                    acc = acc + acc.at[lane ^ sh].get(mode="promise_in_bounds")
                pe = jnp.exp(acc) * valid
                for k in range(8):
                    sb[e, pl.ds(k * 16, 16)] = lrow[k] * pe
                szv = jnp.where(lane == j, pe, szv)
            scz[pl.ds(g * 16, 16)] = szv

        pltpu.async_copy(scz, den_sh.at[idd], sem_z, add=True)
        pltpu.async_copy(sb, out_sh.at[idd], sem_sc, add=True)

    @pl.loop(0, TMAX // 2)
    def _(i):
        phase(2 * i, 0)
        phase(2 * i + 1, 1)

    wait_gather(0)
    wait_idx(1)
    wait_rowscatter()
    wait_zscatter(0)
    wait_zscatter(1)
    plsc.subcore_barrier()

    pltpu.sync_copy(den_sh.at[pl.ds(sid * 640, 640)],
                    dden_hbm.at[pl.ds(cid * NPAD + sid * 640, 640)])
    for t in range(10):
        pltpu.sync_copy(out_sh.at[pl.ds(sid * 640 + t * C, C)],
                        op_hbm.at[pl.ds(cid * NPAD + sid * 640 + t * C, C)])


def _k2(fs, fd, v1d, src, dst):
    mesh = plsc.VectorSubcoreMesh(core_axis_name="c", subcore_axis_name="s")
    return pl.kernel(
        _k2_body,
        out_type=[
            jax.ShapeDtypeStruct((NC * NPAD, D), jnp.float32),
            jax.ShapeDtypeStruct((NC * NPAD,), jnp.float32),
        ],
        mesh=mesh,
        scratch_types=[
            pltpu.VMEM((C,), jnp.int32),
            pltpu.VMEM((C,), jnp.int32),
            pltpu.VMEM((C,), jnp.int32),
            pltpu.VMEM((C,), jnp.int32),
            pltpu.VMEM((C,), jnp.int32),
            pltpu.VMEM((C,), jnp.int32),
            pltpu.VMEM((C, D), jnp.float32),
            pltpu.VMEM((C, D), jnp.float32),
            pltpu.VMEM((C, D), jnp.float32),
            pltpu.VMEM((C, D), jnp.float32),
            pltpu.VMEM((C, D), jnp.float32),
            pltpu.VMEM((C,), jnp.float32),
            pltpu.VMEM((C,), jnp.float32),
            pltpu.VMEM((D,), jnp.float32),
            pltpu.VMEM_SHARED((NPAD,), jnp.float32),
            pltpu.VMEM_SHARED((NPAD, D), jnp.float32),
            pltpu.SemaphoreType.DMA,
            pltpu.SemaphoreType.DMA,
            pltpu.SemaphoreType.DMA,
            pltpu.SemaphoreType.DMA,
            pltpu.SemaphoreType.DMA,
            pltpu.SemaphoreType.DMA,
            pltpu.SemaphoreType.DMA,
        ],
    )(fs, fd, v1d, src, dst)


def _k3_body(op_hbm, dden_hbm, fo_hbm,
             a0a, a1a, a0b, a1b, d0, d1, inv, sema, semb, semo):
    cid = lax.axis_index("c")
    sid = lax.axis_index("s")
    wid = sid * NC + cid
    r0 = wid * 320

    abufs = ((a0a, a1a, sema), (a0b, a1b, semb))

    def issue(t):
        a0, a1, sem = abufs[t % 2]
        rb = r0 + t * 64
        pltpu.async_copy(op_hbm.at[pl.ds(rb, 64)], a0, sem)
        pltpu.async_copy(op_hbm.at[pl.ds(NPAD + rb, 64)], a1, sem)

    def wait(t):
        a0, a1, sem = abufs[t % 2]
        pltpu.make_async_copy(op_hbm.at[pl.ds(0, 64)], a0, sem).wait()
        pltpu.make_async_copy(op_hbm.at[pl.ds(0, 64)], a1, sem).wait()

    def wait_store(t):
        a0 = abufs[t % 2][0]
        pltpu.make_async_copy(a0, fo_hbm.at[pl.ds(0, 64)], semo).wait()

    issue(0)
    pltpu.sync_copy(dden_hbm.at[pl.ds(r0, 320)], d0)
    pltpu.sync_copy(dden_hbm.at[pl.ds(NPAD + r0, 320)], d1)

    @pl.loop(0, 20)
    def _(i):
        s = pl.ds(i * 16, 16)
        inv[s] = 1.0 / (d0[s] + d1[s] + EPS)

    for t in range(5):
        a0, a1, sem = abufs[t % 2]
        wait(t)
        if t < 4:
            issue(t + 1)
        if t >= 2:
            wait_store(t - 2)

        @pl.loop(0, 4)
        def _(g):
            inv16 = inv[pl.ds(t * 64 + g * 16, 16)]
            for j in range(16):
                e = g * 16 + j
                a = inv16[j]
                for k in range(8):
                    s = pl.ds(k * 16, 16)
                    a0[e, s] = (a0[e, s] + a1[e, s]) * a

        pltpu.async_copy(a0, fo_hbm.at[pl.ds(r0 + t * 64, 64)], semo)

    wait_store(3)
    wait_store(4)


def _k3(op, dden):
    mesh = plsc.VectorSubcoreMesh(core_axis_name="c", subcore_axis_name="s")
    return pl.kernel(
        _k3_body,
        out_type=jax.ShapeDtypeStruct((NPAD, D), jnp.float32),
        mesh=mesh,
        scratch_types=[
            pltpu.VMEM((64, D), jnp.float32),
            pltpu.VMEM((64, D), jnp.float32),
            pltpu.VMEM((64, D), jnp.float32),
            pltpu.VMEM((64, D), jnp.float32),
            pltpu.VMEM((320,), jnp.float32),
            pltpu.VMEM((320,), jnp.float32),
            pltpu.VMEM((320,), jnp.float32),
            pltpu.SemaphoreType.DMA,
            pltpu.SemaphoreType.DMA,
            pltpu.SemaphoreType.DMA,
        ],
    )(op, dden)


def kernel(x, edge_index, W_src, b_src, W_dst, b_dst, attn, W_el, b_el):
    src = edge_index[0]
    dst = edge_index[1]
    attn_r = attn.reshape(1, D)
    fs, fd, v = _k1(x, W_src, b_src.reshape(1, D), W_dst, b_dst.reshape(1, D),
                    attn_r, W_el)
    op, dden = _k2(fs, fd, v.reshape(D), src, dst)
    fo = _k3(op, dden)
    return fo[:N].reshape(N, 1, D)
